# baseline (speedup 1.0000x reference)
import functools

import jax
import jax.numpy as jnp
from jax import lax
from jax.experimental import pallas as pl
from jax.experimental.pallas import tpu as pltpu
from jax.experimental.pallas import tpu_sc as plsc

_SC_CORES = 2
_SC_SUBCORES = 16
_SC_WORKERS = _SC_CORES * _SC_SUBCORES


def _vq_block(z_ref, ets_ref, idx_ref, loss_ref, esq_ref):
    i = pl.program_id(0)
    K = ets_ref.shape[1]

    @pl.when(i == 0)
    def _():
        et2 = ets_ref[...]
        esq_ref[...] = 0.25 * jnp.sum(et2 * et2, axis=0, keepdims=True)
        loss_ref[...] = jnp.zeros_like(loss_ref)

    zb = z_ref[...]
    dot = jnp.dot(zb, ets_ref[...], preferred_element_type=jnp.float32)
    zsq = jnp.sum(zb * zb, axis=1, keepdims=True)
    dist = (zsq + dot) + esq_ref[...]
    minv = jnp.min(dist, axis=1, keepdims=True)
    iota = lax.broadcasted_iota(jnp.int32, dist.shape, 1)
    idx = jnp.min(jnp.where(dist == minv, iota, K), axis=1)
    idx_ref[0, 0, :] = idx
    loss_ref[...] += jnp.sum(minv).reshape(1, 1)


def _tc_argmin(flat, ets):
    N, D = flat.shape
    K = ets.shape[1]
    BLK = 512
    NB = N // BLK
    return pl.pallas_call(
        _vq_block,
        grid=(NB,),
        in_specs=[
            pl.BlockSpec((BLK, D), lambda i: (i, 0)),
            pl.BlockSpec((D, K), lambda i: (0, 0)),
        ],
        out_specs=[
            pl.BlockSpec((1, 1, BLK), lambda i: (i, 0, 0)),
            pl.BlockSpec((1, 1), lambda i: (0, 0)),
        ],
        out_shape=[
            jax.ShapeDtypeStruct((NB, 1, BLK), jnp.int32),
            jax.ShapeDtypeStruct((1, 1), jnp.float32),
        ],
        scratch_shapes=[pltpu.VMEM((1, K), jnp.float32)],
    )(flat, ets)


def _sc_gather_body(n_rows, chunk, table_hbm, idx_hbm, out_hbm,
                    idx_v, rows_v, sem, sem_out):
    per_w = n_rows // _SC_WORKERS
    wid = lax.axis_index("s") * _SC_CORES + lax.axis_index("c")
    base = wid * per_w
    pltpu.sync_copy(idx_hbm.at[pl.ds(base, per_w)], idx_v)
    nchunks = per_w // chunk
    for j in range(nchunks):
        pltpu.async_copy(
            table_hbm.at[idx_v.at[pl.ds(j * chunk, chunk)]],
            rows_v.at[pl.ds(j * chunk, chunk)],
            sem,
        )
    for j in range(nchunks):
        pltpu.make_async_copy(
            table_hbm.at[idx_v.at[pl.ds(j * chunk, chunk)]],
            rows_v.at[pl.ds(j * chunk, chunk)],
            sem,
        ).wait()
        pltpu.async_copy(
            rows_v.at[pl.ds(j * chunk, chunk)],
            out_hbm.at[pl.ds(base + j * chunk, chunk)],
            sem_out,
        )
    for j in range(nchunks):
        pltpu.make_async_copy(
            rows_v.at[pl.ds(j * chunk, chunk)],
            out_hbm.at[pl.ds(base + j * chunk, chunk)],
            sem_out,
        ).wait()


def _sc_gather(table, idx_flat):
    K, D = table.shape
    N = idx_flat.shape[0]
    per_w = N // _SC_WORKERS
    chunk = 96
    mesh = plsc.VectorSubcoreMesh(core_axis_name="c", subcore_axis_name="s")
    body = functools.partial(_sc_gather_body, N, chunk)
    return pl.kernel(
        body,
        out_type=jax.ShapeDtypeStruct((N, D), jnp.float32),
        mesh=mesh,
        scratch_types=[
            pltpu.VMEM((per_w,), jnp.int32),
            pltpu.VMEM((per_w, D), jnp.float32),
            pltpu.SemaphoreType.DMA,
            pltpu.SemaphoreType.DMA,
        ],
    )(table, idx_flat)


def kernel(z, embeddings):
    B, T, D = z.shape
    N = B * T
    flat = z.reshape(N, D)
    ets = -2.0 * embeddings.T

    idx3, loss_sum = _tc_argmin(flat, ets)
    emb = _sc_gather(embeddings, idx3.reshape(N))

    encoding_indices = idx3.reshape(B, T)
    emb = emb.reshape(B, T, D)
    commitment_loss = 0.25 * loss_sum[0, 0] / (N * D)
    return emb, encoding_indices, commitment_loss


# speedup vs baseline: 2.0619x; 2.0619x over previous
import jax
import jax.numpy as jnp
from jax import lax
from jax.experimental import pallas as pl
from jax.experimental.pallas import tpu as pltpu


def _vq_block(z_ref, ets_ref, eb_ref, idx_ref, emb_ref, loss_ref, esq_ref):
    i = pl.program_id(0)
    K = ets_ref.shape[1]

    @pl.when(i == 0)
    def _():
        et2 = ets_ref[...]
        esq_ref[...] = 0.25 * jnp.sum(et2 * et2, axis=0, keepdims=True)
        loss_ref[...] = jnp.zeros_like(loss_ref)

    zb = z_ref[...]
    dot = jnp.dot(zb, ets_ref[...], preferred_element_type=jnp.float32)
    zsq = jnp.sum(zb * zb, axis=1, keepdims=True)
    dist = (zsq + dot) + esq_ref[...]
    minv = jnp.min(dist, axis=1, keepdims=True)
    iota = lax.broadcasted_iota(jnp.int32, (1, K), 1).astype(jnp.float32)
    masked = jnp.where(dist == minv, iota, jnp.float32(K))
    idxf = jnp.min(masked, axis=1, keepdims=True)
    idx_ref[0, 0, :] = idxf[:, 0].astype(jnp.int32)
    onehot = jnp.where(masked == idxf, jnp.float32(1), jnp.float32(0))
    emb_ref[...] = jnp.dot(onehot, eb_ref[...],
                           preferred_element_type=jnp.float32)
    loss_ref[...] += jnp.sum(minv).reshape(1, 1)


def kernel(z, embeddings):
    B, T, D = z.shape
    N = B * T
    K = embeddings.shape[0]
    BLK = 1024
    NB = N // BLK
    flat = z.reshape(N, D)
    ets = -2.0 * embeddings.T

    idx3, emb, loss_sum = pl.pallas_call(
        _vq_block,
        grid=(NB,),
        in_specs=[
            pl.BlockSpec((BLK, D), lambda i: (i, 0)),
            pl.BlockSpec((D, K), lambda i: (0, 0)),
            pl.BlockSpec((K, D), lambda i: (0, 0)),
        ],
        out_specs=[
            pl.BlockSpec((1, 1, BLK), lambda i: (i, 0, 0)),
            pl.BlockSpec((BLK, D), lambda i: (i, 0)),
            pl.BlockSpec((1, 1), lambda i: (0, 0)),
        ],
        out_shape=[
            jax.ShapeDtypeStruct((NB, 1, BLK), jnp.int32),
            jax.ShapeDtypeStruct((N, D), jnp.float32),
            jax.ShapeDtypeStruct((1, 1), jnp.float32),
        ],
        scratch_shapes=[pltpu.VMEM((1, K), jnp.float32)],
    )(flat, ets, embeddings)

    encoding_indices = idx3.reshape(B, T)
    emb = emb.reshape(B, T, D)
    commitment_loss = 0.25 * loss_sum[0, 0] / (N * D)
    return emb, encoding_indices, commitment_loss
